# Initial kernel scaffold; baseline (speedup 1.0000x reference)
import jax
import jax.numpy as jnp
from jax.experimental import pallas as pl


def kernel(difference, weight, epoch):
    raise NotImplementedError("write your pallas kernel here")


# speedup vs baseline: 1.2237x; 1.2237x over previous
import functools

import jax
import jax.numpy as jnp
from jax import lax
from jax.experimental import pallas as pl
from jax.experimental.pallas import tpu as pltpu
from jax.experimental.pallas import tpu_sc as plsc

ROWS = 64
COLS = 8192
LANES = 16
NUM_CORES = 2
NUM_SUBCORES = 16
NUM_WORKERS = NUM_CORES * NUM_SUBCORES
ROWS_PER_WORKER = ROWS // NUM_WORKERS
CHUNKS = COLS // LANES


def _row_top3(dbuf, r, lane):
    def scan_body(i, carry):
        m1, m2, m3, i1, i2, i3, idx = carry
        v = jnp.abs(dbuf[r, pl.ds(i * LANES, LANES)])
        gt1 = v > m1
        gt2 = v > m2
        gt3 = v > m3
        nm1 = jnp.where(gt1, v, m1)
        ni1 = jnp.where(gt1, idx, i1)
        nm2 = jnp.where(gt1, m1, jnp.where(gt2, v, m2))
        ni2 = jnp.where(gt1, i1, jnp.where(gt2, idx, i2))
        nm3 = jnp.where(gt2, m2, jnp.where(gt3, v, m3))
        ni3 = jnp.where(gt2, i2, jnp.where(gt3, idx, i3))
        return nm1, nm2, nm3, ni1, ni2, ni3, idx + LANES

    neg = jnp.full((LANES,), -1.0, jnp.float32)
    zero_i = jnp.zeros((LANES,), jnp.int32)
    m1, m2, m3, i1, i2, i3, _ = lax.fori_loop(
        0, CHUNKS, scan_body, (neg, neg, neg, zero_i, zero_i, zero_i, lane)
    )

    winners = []
    for _ in range(3):
        g = jnp.max(m1)
        elig = m1 == g
        cand = jnp.where(elig, i1, COLS)
        w = jnp.min(cand)
        winners.append(w)
        sel = elig & (i1 == w)
        m1 = jnp.where(sel, m2, m1)
        i1 = jnp.where(sel, i2, i1)
        m2 = jnp.where(sel, m3, m2)
        i2 = jnp.where(sel, i3, i2)
        m3 = jnp.where(sel, -1.0, m3)
    return winners


def _body(diff_hbm, w_hbm, gate_hbm, out_hbm, dbuf, obuf, gbuf, sd, sw, sg):
    wid = lax.axis_index("s") * NUM_CORES + lax.axis_index("c")
    r0 = wid * ROWS_PER_WORKER
    cp_d = pltpu.async_copy(diff_hbm.at[pl.ds(r0, ROWS_PER_WORKER)], dbuf, sd)
    cp_w = pltpu.async_copy(w_hbm.at[pl.ds(r0, ROWS_PER_WORKER)], obuf, sw)
    cp_g = pltpu.async_copy(gate_hbm, gbuf, sg)
    cp_d.wait()

    lane = lax.iota(jnp.int32, LANES)
    row_winners = [_row_top3(dbuf, r, lane) for r in range(ROWS_PER_WORKER)]

    cp_w.wait()
    cp_g.wait()
    gate = gbuf[...]
    mask = lane < 3
    zero_i = jnp.zeros((LANES,), jnp.int32)
    for r, (w0, w1, w2) in enumerate(row_winners):
        idxv = jnp.where(
            lane == 0, w0, jnp.where(lane == 1, w1, jnp.where(lane == 2, w2, zero_i))
        )
        rowv = jnp.full((LANES,), r, jnp.int32)
        plsc.addupdate_scatter(obuf, [rowv, idxv], gate, mask=mask)
    pltpu.sync_copy(obuf, out_hbm.at[pl.ds(r0, ROWS_PER_WORKER)])


@jax.jit
def _top3_sc(difference, weight, gate):
    mesh = plsc.VectorSubcoreMesh(
        core_axis_name="c",
        subcore_axis_name="s",
        num_cores=NUM_CORES,
        num_subcores=NUM_SUBCORES,
    )
    fn = pl.kernel(
        _body,
        out_type=jax.ShapeDtypeStruct((ROWS, COLS), jnp.float32),
        mesh=mesh,
        scratch_types=[
            pltpu.VMEM((ROWS_PER_WORKER, COLS), jnp.float32),
            pltpu.VMEM((ROWS_PER_WORKER, COLS), jnp.float32),
            pltpu.VMEM((LANES,), jnp.float32),
            pltpu.SemaphoreType.DMA,
            pltpu.SemaphoreType.DMA,
            pltpu.SemaphoreType.DMA,
        ],
        compiler_params=pltpu.CompilerParams(needs_layout_passes=False),
    )
    return fn(difference, weight, gate)


def kernel(difference, weight, epoch):
    epoch_i = jnp.asarray(epoch, jnp.int32)
    cond = (1000 < epoch_i) & (epoch_i < 18000) & (epoch_i % 200 == 0)
    gate = jnp.where(cond, jnp.float32(1.0), jnp.float32(0.0))
    gate_v = jnp.broadcast_to(gate, (LANES,))
    return _top3_sc(difference, weight, gate_v)
